# baseline (speedup 1.0000x reference)
import functools

import jax
import jax.numpy as jnp
from jax.experimental import pallas as pl
from jax.experimental.pallas import tpu as pltpu

B = 8
C = 192
HW = 224 * 224
ROWS = B * C


def _copy_body(perm_ref, in_ref, out_ref):
    out_ref[...] = in_ref[...]


@jax.jit
def _tc_permute(xf, perm):
    grid_spec = pltpu.PrefetchScalarGridSpec(
        num_scalar_prefetch=1,
        grid=(ROWS,),
        in_specs=[
            pl.BlockSpec(
                (1, 392, 128),
                lambda i, perm_ref: ((i // C) * C + perm_ref[i % C], 0, 0),
            ),
        ],
        out_specs=pl.BlockSpec((1, 392, 128), lambda i, perm_ref: (i, 0, 0)),
    )
    return pl.pallas_call(
        _copy_body,
        grid_spec=grid_spec,
        out_shape=jax.ShapeDtypeStruct((ROWS, 392, 128), jnp.float32),
        compiler_params=pltpu.CompilerParams(
            dimension_semantics=("arbitrary",),
        ),
    )(perm, xf)


def kernel(x, permutation):
    xf = x.reshape(ROWS, 392, 128)
    perm = permutation.astype(jnp.int32)
    out = _tc_permute(xf, perm)
    return out.reshape(B, C, 224, 224)


# speedup vs baseline: 1.0546x; 1.0546x over previous
import functools

import jax
import jax.numpy as jnp
from jax import lax
from jax.experimental import pallas as pl
from jax.experimental.pallas import tpu as pltpu

B = 8
C = 192
HW = 224 * 224
ROWS = B * C
K = 8


def _dma_body(perm_ref, x_ref, out_ref, buf, gsem, psem):
    def wait_put(slot):
        pltpu.make_async_copy(
            buf.at[pl.ds(slot, 1)], out_ref.at[pl.ds(0, 1)], psem.at[slot]
        ).wait()

    def wait_gather(slot):
        pltpu.make_async_copy(
            x_ref.at[pl.ds(0, 1)], buf.at[pl.ds(slot, 1)], gsem.at[slot]
        ).wait()

    def start_put(j, slot):
        pltpu.make_async_copy(
            buf.at[pl.ds(slot, 1)], out_ref.at[pl.ds(j, 1)], psem.at[slot]
        ).start()

    def body(i, _):
        slot = lax.rem(i, K)

        @pl.when(i >= K)
        def _():
            wait_put(slot)

        src = (i // C) * C + perm_ref[lax.rem(i, C)]
        pltpu.make_async_copy(
            x_ref.at[pl.ds(src, 1)], buf.at[pl.ds(slot, 1)], gsem.at[slot]
        ).start()

        @pl.when(i >= 1)
        def _():
            pslot = lax.rem(i - 1, K)
            wait_gather(pslot)
            start_put(i - 1, pslot)

        return 0

    lax.fori_loop(0, ROWS, body, 0)
    last = ROWS - 1
    wait_gather(last % K)
    start_put(last, last % K)
    for s in range(K):
        wait_put((last - s) % K)


@jax.jit
def _tc_permute(xf, perm):
    grid_spec = pltpu.PrefetchScalarGridSpec(
        num_scalar_prefetch=1,
        grid=(1,),
        in_specs=[pl.BlockSpec(memory_space=pl.ANY)],
        out_specs=pl.BlockSpec(memory_space=pl.ANY),
        scratch_shapes=[
            pltpu.VMEM((K, 392, 128), jnp.float32),
            pltpu.SemaphoreType.DMA((K,)),
            pltpu.SemaphoreType.DMA((K,)),
        ],
    )
    return pl.pallas_call(
        _dma_body,
        grid_spec=grid_spec,
        out_shape=jax.ShapeDtypeStruct((ROWS, 392, 128), jnp.float32),
    )(perm, xf)


def kernel(x, permutation):
    xf = x.reshape(ROWS, 392, 128)
    perm = permutation.astype(jnp.int32)
    out = _tc_permute(xf, perm)
    return out.reshape(B, C, 224, 224)
